# baseline (speedup 1.0000x reference)
import functools

import jax
import jax.numpy as jnp
from jax.experimental import pallas as pl
from jax.experimental.pallas import tpu as pltpu

_RB = 8
_GW = 128
_NREF = 8


def _body(tstart_ref, p_ref, *refs, nsteps, bsz):
    (x_refs, (tmod_ref, t_ref), (out1_ref, out5_ref),
     (v_scr, x_scr, rank_scr, sem)) = (refs[:_NREF], refs[_NREF:_NREF + 2],
                                       refs[_NREF + 2:_NREF + 4],
                                       refs[_NREF + 4:])
    j = pl.program_id(0)

    @pl.when(j == 0)
    def _gather():
        copies = [
            pltpu.make_async_copy(
                p_ref.at[pl.ds(8 * (r // 8), 8),
                         pl.ds(pl.multiple_of(tstart_ref[r], _GW), _GW)],
                x_scr.at[r],
                sem,
            )
            for r in range(bsz)
        ]
        for c in copies:
            c.start()
        for c in copies:
            c.wait()
        sub = jax.lax.broadcasted_iota(jnp.int32, (bsz, 8, _GW), 1)
        rmod = jax.lax.broadcasted_iota(jnp.int32, (bsz, 8, _GW), 0) % 8
        lane = jax.lax.broadcasted_iota(jnp.int32, (bsz, 8, _GW), 2)
        sel = jnp.where((sub == rmod) & (lane == tmod_ref[...]),
                        x_scr[...], 0.0)
        v_scr[...] = jnp.sum(sel, axis=(1, 2)).reshape(bsz, 1)

    for r, xr in enumerate(x_refs):
        g = j * _NREF + r
        x = xr[...]
        v = v_scr[pl.ds(g * _RB, _RB), :]
        t = t_ref[pl.ds(g * _RB, _RB), :]
        lane = jax.lax.broadcasted_iota(jnp.int32, x.shape, 1)
        beat = (x > v) | ((x == v) & (lane < t))
        rank_scr[pl.ds(g * _RB, _RB), :] = jnp.sum(beat.astype(jnp.int32),
                                                   axis=1, keepdims=True)

    @pl.when(j == nsteps - 1)
    def _fin():
        rank = rank_scr[...]
        scale = 100.0 / bsz
        out1_ref[...] = jnp.sum((rank < 1).astype(jnp.float32),
                                axis=(0, 1), keepdims=True) * scale
        out5_ref[...] = jnp.sum((rank < 5).astype(jnp.float32),
                                axis=(0, 1), keepdims=True) * scale


def kernel(preds, targets):
    bsz, n = preds.shape
    t32 = targets.astype(jnp.int32)
    tstart = (t32 // _GW) * _GW
    tmod = (t32 % _GW).reshape(bsz, 1, 1)

    nsteps = bsz // (_RB * _NREF)
    out1, out5 = pl.pallas_call(
        functools.partial(_body, nsteps=nsteps, bsz=bsz),
        grid_spec=pltpu.PrefetchScalarGridSpec(
            num_scalar_prefetch=1,
            grid=(nsteps,),
            in_specs=[
                pl.BlockSpec(memory_space=pl.ANY),
            ] + [
                pl.BlockSpec((_RB, n), lambda j, s, r=r: (j * _NREF + r, 0))
                for r in range(_NREF)
            ] + [
                pl.BlockSpec((bsz, 1, 1), lambda j, s: (0, 0, 0)),
                pl.BlockSpec((bsz, 1), lambda j, s: (0, 0)),
            ],
            out_specs=[
                pl.BlockSpec((1, 1), lambda j, s: (0, 0)),
                pl.BlockSpec((1, 1), lambda j, s: (0, 0)),
            ],
            scratch_shapes=[
                pltpu.VMEM((bsz, 1), jnp.float32),
                pltpu.VMEM((bsz, 8, _GW), jnp.float32),
                pltpu.VMEM((bsz, 1), jnp.int32),
                pltpu.SemaphoreType.DMA,
            ],
        ),
        out_shape=[jax.ShapeDtypeStruct((1, 1), jnp.float32)] * 2,
    )(tstart, preds, *([preds] * _NREF), tmod, t32.reshape(bsz, 1))

    return (out1.reshape(1), out5.reshape(1))


# speedup vs baseline: 1.0827x; 1.0827x over previous
import functools

import jax
import jax.numpy as jnp
from jax.experimental import pallas as pl
from jax.experimental.pallas import tpu as pltpu

_RB = 8
_GW = 128
_NREF = 4


def _body(tstart_ref, p_ref, *refs, nsteps, bsz):
    (x_refs, (tmod_ref, t_ref), (out1_ref, out5_ref),
     (v_scr, x_scr, rank_scr, sem)) = (refs[:_NREF], refs[_NREF:_NREF + 2],
                                       refs[_NREF + 2:_NREF + 4],
                                       refs[_NREF + 4:])
    j = pl.program_id(0)

    @pl.when(j == 0)
    def _gather():
        copies = [
            pltpu.make_async_copy(
                p_ref.at[pl.ds(8 * (r // 8), 8),
                         pl.ds(pl.multiple_of(tstart_ref[r], _GW), _GW)],
                x_scr.at[r],
                sem,
            )
            for r in range(bsz)
        ]
        for c in copies:
            c.start()
        for c in copies:
            c.wait()
        sub = jax.lax.broadcasted_iota(jnp.int32, (bsz, 8, _GW), 1)
        rmod = jax.lax.broadcasted_iota(jnp.int32, (bsz, 8, _GW), 0) % 8
        lane = jax.lax.broadcasted_iota(jnp.int32, (bsz, 8, _GW), 2)
        sel = jnp.where((sub == rmod) & (lane == tmod_ref[...]),
                        x_scr[...], 0.0)
        v_scr[...] = jnp.sum(sel, axis=(1, 2)).reshape(bsz, 1)

    for r, xr in enumerate(x_refs):
        g = j * _NREF + r
        x = xr[...]
        v = v_scr[pl.ds(g * _RB, _RB), :]
        t = t_ref[pl.ds(g * _RB, _RB), :]
        lane = jax.lax.broadcasted_iota(jnp.int32, x.shape, 1)
        beat = (x > v) | ((x == v) & (lane < t))
        rank_scr[pl.ds(g * _RB, _RB), :] = jnp.sum(beat.astype(jnp.int32),
                                                   axis=1, keepdims=True)

    @pl.when(j == nsteps - 1)
    def _fin():
        rank = rank_scr[...]
        scale = 100.0 / bsz
        out1_ref[...] = jnp.sum((rank < 1).astype(jnp.float32),
                                axis=(0, 1), keepdims=True) * scale
        out5_ref[...] = jnp.sum((rank < 5).astype(jnp.float32),
                                axis=(0, 1), keepdims=True) * scale


def kernel(preds, targets):
    bsz, n = preds.shape
    t32 = targets.astype(jnp.int32)
    tstart = (t32 // _GW) * _GW
    tmod = (t32 % _GW).reshape(bsz, 1, 1)

    nsteps = bsz // (_RB * _NREF)
    out1, out5 = pl.pallas_call(
        functools.partial(_body, nsteps=nsteps, bsz=bsz),
        grid_spec=pltpu.PrefetchScalarGridSpec(
            num_scalar_prefetch=1,
            grid=(nsteps,),
            in_specs=[
                pl.BlockSpec(memory_space=pl.ANY),
            ] + [
                pl.BlockSpec((_RB, n), lambda j, s, r=r: (j * _NREF + r, 0))
                for r in range(_NREF)
            ] + [
                pl.BlockSpec((bsz, 1, 1), lambda j, s: (0, 0, 0)),
                pl.BlockSpec((bsz, 1), lambda j, s: (0, 0)),
            ],
            out_specs=[
                pl.BlockSpec((1, 1), lambda j, s: (0, 0)),
                pl.BlockSpec((1, 1), lambda j, s: (0, 0)),
            ],
            scratch_shapes=[
                pltpu.VMEM((bsz, 1), jnp.float32),
                pltpu.VMEM((bsz, 8, _GW), jnp.float32),
                pltpu.VMEM((bsz, 1), jnp.int32),
                pltpu.SemaphoreType.DMA,
            ],
        ),
        out_shape=[jax.ShapeDtypeStruct((1, 1), jnp.float32)] * 2,
    )(tstart, preds, *([preds] * _NREF), tmod, t32.reshape(bsz, 1))

    return (out1.reshape(1), out5.reshape(1))
